# baseline (speedup 1.0000x reference)
import functools

import jax
import jax.numpy as jnp
from jax import lax
from jax.experimental import pallas as pl
from jax.experimental.pallas import tpu as pltpu
from jax.experimental.pallas import tpu_sc as plsc

NUM_WORKERS = 32
NSHIFT = 128
LAG = 2


def _expand_kernel(n, nh, row_words, classes_per_worker, i_per_class):
    mesh = plsc.VectorSubcoreMesh(core_axis_name="c", subcore_axis_name="s")

    @functools.partial(
        pl.kernel,
        out_type=jax.ShapeDtypeStruct((nh, n * n), jnp.float32),
        mesh=mesh,
        scratch_types=[
            pltpu.VMEM((nh, row_words), jnp.float32),
            pltpu.SemaphoreType.DMA,
        ],
    )
    def expand(padded_hbm, out_hbm, buf, sem):
        wid = lax.axis_index("s") * 2 + lax.axis_index("c")

        def drain_one():
            pltpu.make_async_copy(
                padded_hbm.at[0, :, pl.ds(0, n)],
                buf.at[:, pl.ds(0, n)], sem).wait()

        for cc in range(classes_per_worker):
            m = wid * classes_per_worker + cc
            r = (m + 1) & (NSHIFT - 1)
            pltpu.sync_copy(padded_hbm.at[m], buf)
            c0 = r + (n - 1) - m

            def v_body(v, carry, m=m, c0=c0):
                col = pl.multiple_of(c0 - NSHIFT * v, NSHIFT)
                dst = pl.multiple_of((m + NSHIFT * v) * n, n)
                pltpu.async_copy(buf.at[:, pl.ds(col, n)],
                                 out_hbm.at[:, pl.ds(dst, n)], sem)

                @pl.when(v >= LAG)
                def _drain_prev():
                    drain_one()

                return carry

            lax.fori_loop(0, i_per_class, v_body, 0)
            for _ in range(LAG):
                drain_one()

    return expand


def kernel(q_len, k_len, bias_table):
    t_rows, nh = bias_table.shape
    n = (t_rows + 1) // 2
    assert n % NSHIFT == 0 and NSHIFT % NUM_WORKERS == 0
    classes_per_worker = NSHIFT // NUM_WORKERS
    i_per_class = n // NSHIFT
    assert i_per_class >= LAG

    base = jnp.asarray(k_len, jnp.int32) - n + jnp.asarray(q_len, jnp.int32) - 1
    u = jnp.arange(2 * n - 1, dtype=jnp.int32)
    ext_idx = jnp.clip(u - (n - 1) + base, 0, t_rows - 1)
    ext_t = bias_table[ext_idx].T

    ext_len = 2 * n - 1
    row_words = ((ext_len + NSHIFT) + NSHIFT - 1) // NSHIFT * NSHIFT
    shift = (jnp.arange(NSHIFT, dtype=jnp.int32) + 1) % NSHIFT
    cols = jnp.arange(row_words, dtype=jnp.int32)
    src = cols[None, :] - shift[:, None]
    valid = (src >= 0) & (src < ext_len)
    padded = jnp.where(valid[:, None, :],
                       ext_t[:, jnp.clip(src, 0, ext_len - 1)].transpose(1, 0, 2),
                       0.0)

    expand = _expand_kernel(n, nh, row_words, classes_per_worker, i_per_class)
    out = expand(padded)
    return out.reshape(1, nh, n, n)


# speedup vs baseline: 16.7589x; 16.7589x over previous
import functools

import jax
import jax.numpy as jnp
from jax import lax
from jax.experimental import pallas as pl
from jax.experimental.pallas import tpu as pltpu
from jax.experimental.pallas import tpu_sc as plsc

NUM_WORKERS = 32
CHUNK = 16
LAG = 2
NSHIFT = 8


def _expand_kernel(n, nh, padded_row_words, rows_per_worker):
    rows_total = nh * n
    mesh = plsc.VectorSubcoreMesh(core_axis_name="c", subcore_axis_name="s")

    @functools.partial(
        pl.kernel,
        out_type=jax.ShapeDtypeStruct((rows_total * n,), jnp.float32),
        mesh=mesh,
        scratch_types=[
            pltpu.VMEM((NSHIFT * padded_row_words,), jnp.float32),
            pltpu.SemaphoreType.DMA,
        ],
    )
    def expand(padded_hbm, out_hbm, buf, sem):
        wid = lax.axis_index("s") * 2 + lax.axis_index("c")
        r0 = wid * rows_per_worker
        r1 = r0 + rows_per_worker
        for t in range(2):
            h = jnp.minimum(r0 // n + t, nh - 1)
            lo = jnp.maximum(r0, h * n)
            hi = jnp.minimum(r1, (h + 1) * n)

            @pl.when((r0 // n + t < nh) & (lo < hi))
            def _per_head(h=h, lo=lo, hi=hi):
                pltpu.sync_copy(padded_hbm.at[h], buf)

                def chunk_wait():
                    pltpu.make_async_copy(
                        padded_hbm.at[h, pl.ds(0, CHUNK * n)],
                        buf.at[pl.ds(0, CHUNK * n)], sem).wait()

                def chunk_body(c, carry):
                    base = lo + c * CHUNK
                    s0 = (n - 1) + h * n - base
                    d0 = base * n
                    for b in range(CHUNK):
                        r = (b + 1) % 8
                        off = pl.multiple_of(
                            s0 + (r * (padded_row_words + 1) - b), 8)
                        dst = pl.multiple_of(d0 + b * n, n)
                        pltpu.async_copy(buf.at[pl.ds(off, n)],
                                         out_hbm.at[pl.ds(dst, n)], sem)

                    @pl.when(c >= LAG)
                    def _drain_prev():
                        chunk_wait()

                    return carry

                lax.fori_loop(0, (hi - lo) // CHUNK, chunk_body, 0)
                for _ in range(LAG):
                    chunk_wait()

    return expand


def kernel(q_len, k_len, bias_table):
    t_rows, nh = bias_table.shape
    n = (t_rows + 1) // 2
    assert (nh * n) % NUM_WORKERS == 0 and n % 8 == 0
    rows_per_worker = nh * n // NUM_WORKERS
    import math
    assert math.gcd(rows_per_worker, n) % CHUNK == 0

    base = jnp.asarray(k_len, jnp.int32) - n + jnp.asarray(q_len, jnp.int32) - 1
    u = jnp.arange(2 * n - 1, dtype=jnp.int32)
    ext_idx = jnp.clip(u - (n - 1) + base, 0, t_rows - 1)
    ext_t = bias_table[ext_idx].T

    ext_len = 2 * n - 1
    row_words = ext_len + NSHIFT
    row_words = ((row_words + 7) // 8) * 8
    shifted = jnp.stack(
        [jnp.pad(ext_t, ((0, 0), (r, row_words - ext_len - r)))
         for r in range(NSHIFT)], axis=1)
    padded = shifted.reshape(nh, NSHIFT * row_words)

    expand = _expand_kernel(n, nh, row_words, rows_per_worker)
    out = expand(padded)
    return out.reshape(1, nh, n, n)
